# baseline (speedup 1.0000x reference)
import jax
import jax.numpy as jnp
from jax.experimental import pallas as pl
from jax.experimental.pallas import tpu as pltpu

_TOP_K = 2
_HID = 1024
_BM_GATE = 512
_BM = 1024


def _gating_kernel(x_ref, gw_ref, gb_ref, idx_ref, xb_ref, acc_ref):
    i = pl.program_id(0)
    n = pl.num_programs(0)
    num_experts = gw_ref.shape[0]
    xb_ref[...] = x_ref[...].astype(jnp.bfloat16)
    logits = jax.lax.dot_general(
        x_ref[...], gw_ref[...],
        dimension_numbers=(((1,), (1,)), ((), ())),
        preferred_element_type=jnp.float32,
    )
    part = jnp.sum(logits, axis=0, keepdims=True)

    @pl.when(i == 0)
    def _():
        acc_ref[:1, :num_experts] = part

    @pl.when(i > 0)
    def _():
        acc_ref[:1, :num_experts] += part

    @pl.when(i == n - 1)
    def _():
        gs = acc_ref[:1, :num_experts] + gb_ref[...]
        iota = jax.lax.broadcasted_iota(jnp.int32, (1, num_experts), 1)
        big = jnp.int32(num_experts)
        m0 = jnp.max(gs)
        i0 = jnp.min(jnp.where(gs == m0, iota, big))
        gs2 = jnp.where(iota == i0, -jnp.inf, gs)
        m1 = jnp.max(gs2)
        i1 = jnp.min(jnp.where(gs2 == m1, iota, big))
        idx_ref[0] = i0
        idx_ref[1] = i1


def _collapse_kernel(idx_ref, w_ref, a_ref, m_ref, acc_ref):
    k = pl.program_id(1)
    p = jax.lax.dot_general(
        w_ref[...].astype(jnp.bfloat16), a_ref[...].astype(jnp.bfloat16),
        dimension_numbers=(((1,), (1,)), ((), ())),
        preferred_element_type=jnp.float32,
    )

    @pl.when(k == 0)
    def _():
        acc_ref[...] = p

    @pl.when(k > 0)
    def _():
        m_ref[...] = (acc_ref[...] + p).astype(jnp.bfloat16)


def _moe_kernel(xb_ref, m_ref, b_ref, o_ref):
    y = jax.lax.dot_general(
        xb_ref[...], m_ref[...],
        dimension_numbers=(((1,), (0,)), ((), ())),
        preferred_element_type=jnp.float32,
    )
    o_ref[...] = y + b_ref[...]


def kernel(x, gating_w, gating_b, weight, agg_w, agg_b):
    batch, in_dim = x.shape
    num_experts = gating_w.shape[0]
    out_dim = agg_w.shape[0]

    gb_total = (gating_b.astype(jnp.float32) * batch).reshape(1, num_experts)

    idx, xb = pl.pallas_call(
        _gating_kernel,
        grid=(batch // _BM_GATE,),
        in_specs=[
            pl.BlockSpec((_BM_GATE, in_dim), lambda i: (i, 0)),
            pl.BlockSpec((num_experts, in_dim), lambda i: (0, 0)),
            pl.BlockSpec((1, num_experts), lambda i: (0, 0)),
        ],
        out_specs=[
            pl.BlockSpec(memory_space=pltpu.SMEM),
            pl.BlockSpec((_BM_GATE, in_dim), lambda i: (i, 0)),
        ],
        out_shape=[
            jax.ShapeDtypeStruct((_TOP_K,), jnp.int32),
            jax.ShapeDtypeStruct((batch, in_dim), jnp.bfloat16),
        ],
        scratch_shapes=[pltpu.VMEM((8, 128), jnp.float32)],
    )(x, gating_w, gb_total)

    bn = out_dim // 2
    collapse_spec = pltpu.PrefetchScalarGridSpec(
        num_scalar_prefetch=1,
        grid=(out_dim // bn, _TOP_K),
        in_specs=[
            pl.BlockSpec((in_dim, _HID), lambda j, k, idx_ref: (0, idx_ref[k])),
            pl.BlockSpec((bn, _HID), lambda j, k, idx_ref: (j, idx_ref[k])),
        ],
        out_specs=pl.BlockSpec((in_dim, bn), lambda j, k, idx_ref: (0, j)),
        scratch_shapes=[pltpu.VMEM((in_dim, bn), jnp.float32)],
    )
    m = pl.pallas_call(
        _collapse_kernel,
        grid_spec=collapse_spec,
        out_shape=jax.ShapeDtypeStruct((in_dim, out_dim), jnp.bfloat16),
    )(idx, weight, agg_w)

    b2 = agg_b.reshape(1, out_dim)
    out = pl.pallas_call(
        _moe_kernel,
        grid=(batch // _BM,),
        in_specs=[
            pl.BlockSpec((_BM, in_dim), lambda i: (i, 0)),
            pl.BlockSpec((in_dim, out_dim), lambda i: (0, 0)),
            pl.BlockSpec((1, out_dim), lambda i: (0, 0)),
        ],
        out_specs=pl.BlockSpec((_BM, out_dim), lambda i: (i, 0)),
        out_shape=jax.ShapeDtypeStruct((batch, out_dim), jnp.float32),
        compiler_params=pltpu.CompilerParams(
            dimension_semantics=("arbitrary",),
        ),
    )(xb, m, b2)
    return out


# speedup vs baseline: 1.0631x; 1.0631x over previous
import jax
import jax.numpy as jnp
from jax.experimental import pallas as pl
from jax.experimental.pallas import tpu as pltpu

_TOP_K = 2
_HID = 1024
_BM = 512


def _gating_kernel(x_ref, gw_ref, gb_ref, idx_ref, acc_ref):
    i = pl.program_id(0)
    n = pl.num_programs(0)
    num_experts = gw_ref.shape[0]
    logits = jax.lax.dot_general(
        x_ref[...], gw_ref[...],
        dimension_numbers=(((1,), (1,)), ((), ())),
        preferred_element_type=jnp.float32,
    )
    part = jnp.sum(logits, axis=0, keepdims=True)

    @pl.when(i == 0)
    def _():
        acc_ref[:1, :num_experts] = part

    @pl.when(i > 0)
    def _():
        acc_ref[:1, :num_experts] += part

    @pl.when(i == n - 1)
    def _():
        gs = acc_ref[:1, :num_experts] + gb_ref[...]
        iota = jax.lax.broadcasted_iota(jnp.int32, (1, num_experts), 1)
        big = jnp.int32(num_experts)
        m0 = jnp.max(gs)
        i0 = jnp.min(jnp.where(gs == m0, iota, big))
        gs2 = jnp.where(iota == i0, -jnp.inf, gs)
        m1 = jnp.max(gs2)
        i1 = jnp.min(jnp.where(gs2 == m1, iota, big))
        idx_ref[0] = i0
        idx_ref[1] = i1


def _collapse_kernel(idx_ref, w_ref, a_ref, m_ref, acc_ref):
    k = pl.program_id(1)
    p = jax.lax.dot_general(
        w_ref[...].astype(jnp.bfloat16), a_ref[...].astype(jnp.bfloat16),
        dimension_numbers=(((1,), (1,)), ((), ())),
        preferred_element_type=jnp.float32,
    )

    @pl.when(k == 0)
    def _():
        acc_ref[...] = p

    @pl.when(k > 0)
    def _():
        m_ref[...] = (acc_ref[...] + p).astype(jnp.bfloat16)


def _moe_kernel(x_ref, m_ref, b_ref, o_ref):
    xb = x_ref[...].astype(jnp.bfloat16)
    y = jax.lax.dot_general(
        xb, m_ref[...],
        dimension_numbers=(((1,), (0,)), ((), ())),
        preferred_element_type=jnp.float32,
    )
    o_ref[...] = y + b_ref[...]


def kernel(x, gating_w, gating_b, weight, agg_w, agg_b):
    batch, in_dim = x.shape
    num_experts = gating_w.shape[0]
    out_dim = agg_w.shape[0]

    gb_total = (gating_b.astype(jnp.float32) * batch).reshape(1, num_experts)

    idx = pl.pallas_call(
        _gating_kernel,
        grid=(batch // _BM,),
        in_specs=[
            pl.BlockSpec((_BM, in_dim), lambda i: (i, 0)),
            pl.BlockSpec((num_experts, in_dim), lambda i: (0, 0)),
            pl.BlockSpec((1, num_experts), lambda i: (0, 0)),
        ],
        out_specs=pl.BlockSpec(memory_space=pltpu.SMEM),
        out_shape=jax.ShapeDtypeStruct((_TOP_K,), jnp.int32),
        scratch_shapes=[pltpu.VMEM((8, 128), jnp.float32)],
    )(x, gating_w, gb_total)

    bn = out_dim // 2
    collapse_spec = pltpu.PrefetchScalarGridSpec(
        num_scalar_prefetch=1,
        grid=(out_dim // bn, _TOP_K),
        in_specs=[
            pl.BlockSpec((in_dim, _HID), lambda j, k, idx_ref: (0, idx_ref[k])),
            pl.BlockSpec((bn, _HID), lambda j, k, idx_ref: (j, idx_ref[k])),
        ],
        out_specs=pl.BlockSpec((in_dim, bn), lambda j, k, idx_ref: (0, j)),
        scratch_shapes=[pltpu.VMEM((in_dim, bn), jnp.float32)],
    )
    m = pl.pallas_call(
        _collapse_kernel,
        grid_spec=collapse_spec,
        out_shape=jax.ShapeDtypeStruct((in_dim, out_dim), jnp.bfloat16),
        compiler_params=pltpu.CompilerParams(
            vmem_limit_bytes=100 * 1024 * 1024,
        ),
    )(idx, weight, agg_w)

    b2 = agg_b.reshape(1, out_dim)
    out = pl.pallas_call(
        _moe_kernel,
        grid=(batch // _BM,),
        in_specs=[
            pl.BlockSpec((_BM, in_dim), lambda i: (i, 0)),
            pl.BlockSpec((in_dim, out_dim), lambda i: (0, 0)),
            pl.BlockSpec((1, out_dim), lambda i: (0, 0)),
        ],
        out_specs=pl.BlockSpec((_BM, out_dim), lambda i: (i, 0)),
        out_shape=jax.ShapeDtypeStruct((batch, out_dim), jnp.float32),
        compiler_params=pltpu.CompilerParams(
            dimension_semantics=("arbitrary",),
        ),
    )(x, m, b2)
    return out


# speedup vs baseline: 1.0663x; 1.0029x over previous
import jax
import jax.numpy as jnp
from jax.experimental import pallas as pl
from jax.experimental.pallas import tpu as pltpu

_TOP_K = 2
_HID = 1024
_BM = 512


def _gating_kernel(x_ref, gw_ref, gb_ref, idx_ref, acc_ref):
    i = pl.program_id(0)
    n = pl.num_programs(0)
    num_experts = gw_ref.shape[0]
    logits = jax.lax.dot_general(
        x_ref[...], gw_ref[...],
        dimension_numbers=(((1,), (1,)), ((), ())),
        preferred_element_type=jnp.float32,
    )
    part = jnp.sum(logits, axis=0, keepdims=True)

    @pl.when(i == 0)
    def _():
        acc_ref[:1, :num_experts] = part

    @pl.when(i > 0)
    def _():
        acc_ref[:1, :num_experts] += part

    @pl.when(i == n - 1)
    def _():
        gs = acc_ref[:1, :num_experts] + gb_ref[...]
        iota = jax.lax.broadcasted_iota(jnp.int32, (1, num_experts), 1)
        big = jnp.int32(num_experts)
        m0 = jnp.max(gs)
        i0 = jnp.min(jnp.where(gs == m0, iota, big))
        gs2 = jnp.where(iota == i0, -jnp.inf, gs)
        m1 = jnp.max(gs2)
        i1 = jnp.min(jnp.where(gs2 == m1, iota, big))
        idx_ref[0] = i0
        idx_ref[1] = i1


def _collapse_kernel(idx_ref, w_ref, a_ref, m_ref, acc_ref):
    k = pl.program_id(1)
    p = jax.lax.dot_general(
        w_ref[...].astype(jnp.bfloat16), a_ref[...].astype(jnp.bfloat16),
        dimension_numbers=(((1,), (1,)), ((), ())),
        preferred_element_type=jnp.float32,
    )

    @pl.when(k == 0)
    def _():
        acc_ref[...] = p

    @pl.when(k > 0)
    def _():
        m_ref[...] = (acc_ref[...] + p).astype(jnp.bfloat16)


def _moe_kernel(x_ref, m_ref, b_ref, o_ref):
    xb = x_ref[...].astype(jnp.bfloat16)
    hn = m_ref.shape[1] // 2
    y0 = jax.lax.dot_general(
        xb, m_ref[:, :hn],
        dimension_numbers=(((1,), (0,)), ((), ())),
        preferred_element_type=jnp.float32,
    )
    y1 = jax.lax.dot_general(
        xb, m_ref[:, hn:],
        dimension_numbers=(((1,), (0,)), ((), ())),
        preferred_element_type=jnp.float32,
    )
    o_ref[:, :hn] = y0 + b_ref[:, :hn]
    o_ref[:, hn:] = y1 + b_ref[:, hn:]


def kernel(x, gating_w, gating_b, weight, agg_w, agg_b):
    batch, in_dim = x.shape
    num_experts = gating_w.shape[0]
    out_dim = agg_w.shape[0]

    gb_total = (gating_b.astype(jnp.float32) * batch).reshape(1, num_experts)

    idx = pl.pallas_call(
        _gating_kernel,
        grid=(batch // _BM,),
        in_specs=[
            pl.BlockSpec((_BM, in_dim), lambda i: (i, 0)),
            pl.BlockSpec((num_experts, in_dim), lambda i: (0, 0)),
            pl.BlockSpec((1, num_experts), lambda i: (0, 0)),
        ],
        out_specs=pl.BlockSpec(memory_space=pltpu.SMEM),
        out_shape=jax.ShapeDtypeStruct((_TOP_K,), jnp.int32),
        scratch_shapes=[pltpu.VMEM((8, 128), jnp.float32)],
    )(x, gating_w, gb_total)

    bn = out_dim // 2
    collapse_spec = pltpu.PrefetchScalarGridSpec(
        num_scalar_prefetch=1,
        grid=(out_dim // bn, _TOP_K),
        in_specs=[
            pl.BlockSpec((in_dim, _HID), lambda j, k, idx_ref: (0, idx_ref[k])),
            pl.BlockSpec((bn, _HID), lambda j, k, idx_ref: (j, idx_ref[k])),
        ],
        out_specs=pl.BlockSpec((in_dim, bn), lambda j, k, idx_ref: (0, j)),
        scratch_shapes=[pltpu.VMEM((in_dim, bn), jnp.float32)],
    )
    m = pl.pallas_call(
        _collapse_kernel,
        grid_spec=collapse_spec,
        out_shape=jax.ShapeDtypeStruct((in_dim, out_dim), jnp.bfloat16),
        compiler_params=pltpu.CompilerParams(
            vmem_limit_bytes=100 * 1024 * 1024,
        ),
    )(idx, weight, agg_w)

    b2 = agg_b.reshape(1, out_dim)
    out = pl.pallas_call(
        _moe_kernel,
        grid=(batch // _BM,),
        in_specs=[
            pl.BlockSpec((_BM, in_dim), lambda i: (i, 0)),
            pl.BlockSpec((in_dim, out_dim), lambda i: (0, 0)),
            pl.BlockSpec((1, out_dim), lambda i: (0, 0)),
        ],
        out_specs=pl.BlockSpec((_BM, out_dim), lambda i: (i, 0)),
        out_shape=jax.ShapeDtypeStruct((batch, out_dim), jnp.float32),
        compiler_params=pltpu.CompilerParams(
            dimension_semantics=("arbitrary",),
        ),
    )(x, m, b2)
    return out


# speedup vs baseline: 1.0769x; 1.0100x over previous
import jax
import jax.numpy as jnp
from jax.experimental import pallas as pl
from jax.experimental.pallas import tpu as pltpu

_TOP_K = 2
_HID = 1024
_BM = 512


def _gating_kernel(x_ref, gw_ref, gb_ref, idx_ref, acc_ref):
    i = pl.program_id(0)
    n = pl.num_programs(0)
    num_experts = gw_ref.shape[0]
    logits = jax.lax.dot_general(
        x_ref[...], gw_ref[...],
        dimension_numbers=(((1,), (1,)), ((), ())),
        preferred_element_type=jnp.float32,
    )
    part = jnp.sum(logits, axis=0, keepdims=True)

    @pl.when(i == 0)
    def _():
        acc_ref[:1, :num_experts] = part

    @pl.when(i > 0)
    def _():
        acc_ref[:1, :num_experts] += part

    @pl.when(i == n - 1)
    def _():
        gs = acc_ref[:1, :num_experts] + gb_ref[...]
        iota = jax.lax.broadcasted_iota(jnp.int32, (1, num_experts), 1)
        big = jnp.int32(num_experts)
        m0 = jnp.max(gs)
        i0 = jnp.min(jnp.where(gs == m0, iota, big))
        gs2 = jnp.where(iota == i0, -jnp.inf, gs)
        m1 = jnp.max(gs2)
        i1 = jnp.min(jnp.where(gs2 == m1, iota, big))
        idx_ref[0] = i0
        idx_ref[1] = i1


def _collapse_kernel(idx_ref, w_ref, a_ref, m0_ref, m1_ref, wb_ref):
    k = pl.program_id(0)
    j = pl.program_id(1)

    @pl.when(j == 0)
    def _():
        wb_ref[...] = w_ref[...].astype(jnp.bfloat16)

    p = jax.lax.dot_general(
        wb_ref[...], a_ref[...].astype(jnp.bfloat16),
        dimension_numbers=(((1,), (1,)), ((), ())),
        preferred_element_type=jnp.float32,
    )

    @pl.when((k == 0) & (j == 0))
    def _():
        m0_ref[...] = p.astype(jnp.bfloat16)

    @pl.when((k == 0) & (j > 0))
    def _():
        m1_ref[...] = p.astype(jnp.bfloat16)

    @pl.when((k > 0) & (j == 0))
    def _():
        m0_ref[...] = (m0_ref[...].astype(jnp.float32) + p).astype(jnp.bfloat16)

    @pl.when((k > 0) & (j > 0))
    def _():
        m1_ref[...] = (m1_ref[...].astype(jnp.float32) + p).astype(jnp.bfloat16)


def _moe_kernel(x_ref, m0_ref, m1_ref, b_ref, o_ref):
    xb = x_ref[...].astype(jnp.bfloat16)
    hn = m0_ref.shape[1]
    y0 = jax.lax.dot_general(
        xb, m0_ref[...],
        dimension_numbers=(((1,), (0,)), ((), ())),
        preferred_element_type=jnp.float32,
    )
    y1 = jax.lax.dot_general(
        xb, m1_ref[...],
        dimension_numbers=(((1,), (0,)), ((), ())),
        preferred_element_type=jnp.float32,
    )
    o_ref[:, :hn] = y0 + b_ref[:, :hn]
    o_ref[:, hn:] = y1 + b_ref[:, hn:]


def kernel(x, gating_w, gating_b, weight, agg_w, agg_b):
    batch, in_dim = x.shape
    num_experts = gating_w.shape[0]
    out_dim = agg_w.shape[0]

    gb_total = (gating_b.astype(jnp.float32) * batch).reshape(1, num_experts)

    bm_gate = 1024
    idx = pl.pallas_call(
        _gating_kernel,
        grid=(batch // bm_gate,),
        in_specs=[
            pl.BlockSpec((bm_gate, in_dim), lambda i: (i, 0)),
            pl.BlockSpec((num_experts, in_dim), lambda i: (0, 0)),
            pl.BlockSpec((1, num_experts), lambda i: (0, 0)),
        ],
        out_specs=pl.BlockSpec(memory_space=pltpu.SMEM),
        out_shape=jax.ShapeDtypeStruct((_TOP_K,), jnp.int32),
        scratch_shapes=[pltpu.VMEM((8, 128), jnp.float32)],
    )(x, gating_w, gb_total)

    bn = out_dim // 2
    collapse_spec = pltpu.PrefetchScalarGridSpec(
        num_scalar_prefetch=1,
        grid=(_TOP_K, out_dim // bn),
        in_specs=[
            pl.BlockSpec((in_dim, _HID), lambda k, j, idx_ref: (0, idx_ref[k])),
            pl.BlockSpec((bn, _HID), lambda k, j, idx_ref: (j, idx_ref[k])),
        ],
        out_specs=[
            pl.BlockSpec((in_dim, bn), lambda k, j, idx_ref: (0, 0)),
            pl.BlockSpec((in_dim, bn), lambda k, j, idx_ref: (0, 0)),
        ],
        scratch_shapes=[
            pltpu.VMEM((in_dim, _HID), jnp.bfloat16),
        ],
    )
    m0, m1 = pl.pallas_call(
        _collapse_kernel,
        grid_spec=collapse_spec,
        out_shape=[
            jax.ShapeDtypeStruct((in_dim, bn), jnp.bfloat16),
            jax.ShapeDtypeStruct((in_dim, bn), jnp.bfloat16),
        ],
        compiler_params=pltpu.CompilerParams(
            vmem_limit_bytes=63 * 1024 * 1024,
        ),
    )(idx, weight, agg_w)


    b2 = agg_b.reshape(1, out_dim)
    out = pl.pallas_call(
        _moe_kernel,
        grid=(batch // _BM,),
        in_specs=[
            pl.BlockSpec((_BM, in_dim), lambda i: (i, 0)),
            pl.BlockSpec((in_dim, bn), lambda i: (0, 0)),
            pl.BlockSpec((in_dim, bn), lambda i: (0, 0)),
            pl.BlockSpec((1, out_dim), lambda i: (0, 0)),
        ],
        out_specs=pl.BlockSpec((_BM, out_dim), lambda i: (i, 0)),
        out_shape=jax.ShapeDtypeStruct((batch, out_dim), jnp.float32),
        compiler_params=pltpu.CompilerParams(
            dimension_semantics=("arbitrary",),
        ),
    )(x, m0, m1, b2)
    return out
